# baseline (speedup 1.0000x reference)
import functools

import jax
import jax.numpy as jnp
from jax import lax
from jax.experimental import pallas as pl
from jax.experimental.pallas import tpu as pltpu
from jax.experimental.pallas import tpu_sc as plsc

VOCAB = 100000
BATCH = 1024
SEQ = 200
LANES = 16
GROUPS = (SEQ + LANES - 1) // LANES
SEQ_PAD = GROUPS * LANES
NUM_WORKERS = 32
ROWS_PER_W = BATCH // NUM_WORKERS
ZERO_UNROLL = 10

BGROUPS = BATCH // 128
VB = 2176
GV = (VOCAB + VB - 1) // VB
V8 = VOCAB // 8


def _sc_body(tok_hbm, iix_hbm, out_hbm, toks_v, iix_v, hist_v):
    c = lax.axis_index("c")
    s = lax.axis_index("s")
    wid = s * 2 + c
    base = wid * ROWS_PER_W

    pltpu.sync_copy(tok_hbm.at[pl.ds(base, ROWS_PER_W)], toks_v)
    pltpu.sync_copy(iix_hbm, iix_v)

    zeros_f = jnp.zeros((LANES,), jnp.float32)
    ones_f = jnp.ones((LANES,), jnp.float32)

    def _zero(i, carry):
        for u in range(ZERO_UNROLL):
            hist_v[pl.ds((i * ZERO_UNROLL + u) * LANES, LANES)] = zeros_f
        return carry

    lax.fori_loop(0, VOCAB // (LANES * ZERO_UNROLL), _zero, 0)

    iix16 = iix_v[...]

    def _row(r, carry):
        for g in range(GROUPS):
            idx = toks_v[r, pl.ds(g * LANES, LANES)]
            plsc.addupdate_scatter(hist_v, [idx], ones_f)
        plsc.store_scatter(hist_v, [iix16], zeros_f)
        pltpu.sync_copy(hist_v, out_hbm.at[base + r])
        for g in range(GROUPS):
            idx = toks_v[r, pl.ds(g * LANES, LANES)]
            plsc.store_scatter(hist_v, [idx], zeros_f)
        return carry

    lax.fori_loop(0, ROWS_PER_W, _row, 0)


def _tr_body(x_ref, o_ref):
    x = x_ref[...]
    o_ref[...] = x.T.reshape(VB // 8, 1, 8, 128)


def _to_batch_minor(counts_rm):
    counts4 = pl.pallas_call(
        _tr_body,
        grid=(BGROUPS, GV),
        in_specs=[pl.BlockSpec((128, VB), lambda i, j: (i, j))],
        out_specs=pl.BlockSpec((VB // 8, 1, 8, 128), lambda i, j: (j, i, 0, 0)),
        out_shape=jax.ShapeDtypeStruct((V8, BGROUPS, 8, 128), jnp.float32),
    )(counts_rm)
    return counts4.transpose(1, 3, 0, 2).reshape(BATCH, VOCAB)


@jax.jit
def kernel(inputs, ignore_index):
    iix = jnp.asarray(ignore_index, jnp.int32)
    toks = inputs.astype(jnp.int32)
    pad = jnp.full((BATCH, SEQ_PAD - SEQ), iix, jnp.int32)
    toks = jnp.concatenate([toks, pad], axis=1)
    iix_arr = jnp.full((LANES,), iix, jnp.int32)

    mesh = plsc.VectorSubcoreMesh(core_axis_name="c", subcore_axis_name="s")
    counts_rm = pl.kernel(
        _sc_body,
        out_type=jax.ShapeDtypeStruct((BATCH, VOCAB), jnp.float32),
        mesh=mesh,
        scratch_types=[
            pltpu.VMEM((ROWS_PER_W, SEQ_PAD), jnp.int32),
            pltpu.VMEM((LANES,), jnp.int32),
            pltpu.VMEM((VOCAB,), jnp.float32),
        ],
        compiler_params=pltpu.CompilerParams(
            needs_layout_passes=False, use_tc_tiling_on_sc=True
        ),
    )(toks, iix_arr)
    return _to_batch_minor(counts_rm)


# speedup vs baseline: 1.2405x; 1.2405x over previous
import functools

import jax
import jax.numpy as jnp
from jax import lax
from jax.experimental import pallas as pl
from jax.experimental.pallas import tpu as pltpu
from jax.experimental.pallas import tpu_sc as plsc

VOCAB = 100000
BATCH = 1024
SEQ = 200
LANES = 16
GROUPS = (SEQ + LANES - 1) // LANES
SEQ_PAD = GROUPS * LANES
NUM_WORKERS = 32
ROWS_PER_W = BATCH // NUM_WORKERS
ZERO_UNROLL = 10

BGROUPS = BATCH // 128
VB = 4352
GV = (VOCAB + VB - 1) // VB
V8 = VOCAB // 8


def _sc_body(tok_hbm, iix_hbm, out_hbm, toks_v, iix_v, hist_v):
    c = lax.axis_index("c")
    s = lax.axis_index("s")
    wid = s * 2 + c
    base = wid * ROWS_PER_W

    pltpu.sync_copy(tok_hbm.at[pl.ds(base, ROWS_PER_W)], toks_v)
    pltpu.sync_copy(iix_hbm, iix_v)

    zeros_f = jnp.zeros((LANES,), jnp.float32)
    ones_f = jnp.ones((LANES,), jnp.float32)

    def _zero(i, carry):
        for u in range(ZERO_UNROLL):
            hist_v[pl.ds((i * ZERO_UNROLL + u) * LANES, LANES)] = zeros_f
        return carry

    lax.fori_loop(0, VOCAB // (LANES * ZERO_UNROLL), _zero, 0)

    iix16 = iix_v[...]

    def _row(r, carry):
        for g in range(GROUPS):
            idx = toks_v[r, pl.ds(g * LANES, LANES)]
            plsc.addupdate_scatter(hist_v, [idx], ones_f)
        plsc.store_scatter(hist_v, [iix16], zeros_f)
        pltpu.sync_copy(hist_v, out_hbm.at[base + r])
        for g in range(GROUPS):
            idx = toks_v[r, pl.ds(g * LANES, LANES)]
            plsc.store_scatter(hist_v, [idx], zeros_f)
        return carry

    lax.fori_loop(0, ROWS_PER_W, _row, 0)


def _tr_body(x_ref, o_ref):
    x = x_ref[...]
    o_ref[...] = x.T.reshape(VB // 8, 1, 8, 128)


def _to_batch_minor(counts_rm):
    counts4 = pl.pallas_call(
        _tr_body,
        grid=(BGROUPS, GV),
        in_specs=[pl.BlockSpec((128, VB), lambda i, j: (i, j))],
        out_specs=pl.BlockSpec((VB // 8, 1, 8, 128), lambda i, j: (j, i, 0, 0)),
        out_shape=jax.ShapeDtypeStruct((V8, BGROUPS, 8, 128), jnp.float32),
        compiler_params=pltpu.CompilerParams(
            dimension_semantics=("parallel", "parallel")
        ),
    )(counts_rm)
    return counts4.transpose(1, 3, 0, 2).reshape(BATCH, VOCAB)


@jax.jit
def kernel(inputs, ignore_index):
    iix = jnp.asarray(ignore_index, jnp.int32)
    toks = inputs.astype(jnp.int32)
    pad = jnp.full((BATCH, SEQ_PAD - SEQ), iix, jnp.int32)
    toks = jnp.concatenate([toks, pad], axis=1)
    iix_arr = jnp.full((LANES,), iix, jnp.int32)

    mesh = plsc.VectorSubcoreMesh(core_axis_name="c", subcore_axis_name="s")
    counts_rm = pl.kernel(
        _sc_body,
        out_type=jax.ShapeDtypeStruct((BATCH, VOCAB), jnp.float32),
        mesh=mesh,
        scratch_types=[
            pltpu.VMEM((ROWS_PER_W, SEQ_PAD), jnp.int32),
            pltpu.VMEM((LANES,), jnp.int32),
            pltpu.VMEM((VOCAB,), jnp.float32),
        ],
        compiler_params=pltpu.CompilerParams(
            needs_layout_passes=False, use_tc_tiling_on_sc=True
        ),
    )(toks, iix_arr)
    return _to_batch_minor(counts_rm)


# speedup vs baseline: 1.3431x; 1.0828x over previous
import functools

import jax
import jax.numpy as jnp
from jax import lax
from jax.experimental import pallas as pl
from jax.experimental.pallas import tpu as pltpu
from jax.experimental.pallas import tpu_sc as plsc

VOCAB = 100000
BATCH = 1024
SEQ = 200
LANES = 16
GROUPS = (SEQ + LANES - 1) // LANES
SEQ_PAD = GROUPS * LANES
NUM_WORKERS = 32
ROWS_PER_W = BATCH // NUM_WORKERS
ZERO_UNROLL = 10

BGROUPS = BATCH // 128
VB = 8704
GV = (VOCAB + VB - 1) // VB
V8 = VOCAB // 8


def _sc_body(tok_hbm, iix_hbm, out_hbm, toks_v, iix_v, hist_v):
    c = lax.axis_index("c")
    s = lax.axis_index("s")
    wid = s * 2 + c
    base = wid * ROWS_PER_W

    pltpu.sync_copy(tok_hbm.at[pl.ds(base, ROWS_PER_W)], toks_v)
    pltpu.sync_copy(iix_hbm, iix_v)

    zeros_f = jnp.zeros((LANES,), jnp.float32)
    ones_f = jnp.ones((LANES,), jnp.float32)

    def _zero(i, carry):
        for u in range(ZERO_UNROLL):
            hist_v[pl.ds((i * ZERO_UNROLL + u) * LANES, LANES)] = zeros_f
        return carry

    lax.fori_loop(0, VOCAB // (LANES * ZERO_UNROLL), _zero, 0)

    iix16 = iix_v[...]

    def _row(r, carry):
        for g in range(GROUPS):
            idx = toks_v[r, pl.ds(g * LANES, LANES)]
            plsc.addupdate_scatter(hist_v, [idx], ones_f)
        plsc.store_scatter(hist_v, [iix16], zeros_f)
        pltpu.sync_copy(hist_v, out_hbm.at[base + r])
        for g in range(GROUPS):
            idx = toks_v[r, pl.ds(g * LANES, LANES)]
            plsc.store_scatter(hist_v, [idx], zeros_f)
        return carry

    lax.fori_loop(0, ROWS_PER_W, _row, 0)


def _tr_body(x_ref, o_ref):
    x = x_ref[...]
    o_ref[...] = x.T.reshape(VB // 8, 1, 8, 128)


def _to_batch_minor(counts_rm):
    counts4 = pl.pallas_call(
        _tr_body,
        grid=(BGROUPS, GV),
        in_specs=[pl.BlockSpec((128, VB), lambda i, j: (i, j))],
        out_specs=pl.BlockSpec((VB // 8, 1, 8, 128), lambda i, j: (j, i, 0, 0)),
        out_shape=jax.ShapeDtypeStruct((V8, BGROUPS, 8, 128), jnp.float32),
        compiler_params=pltpu.CompilerParams(
            dimension_semantics=("parallel", "parallel")
        ),
    )(counts_rm)
    return counts4.transpose(1, 3, 0, 2).reshape(BATCH, VOCAB)


@jax.jit
def kernel(inputs, ignore_index):
    iix = jnp.asarray(ignore_index, jnp.int32)
    toks = inputs.astype(jnp.int32)
    pad = jnp.full((BATCH, SEQ_PAD - SEQ), iix, jnp.int32)
    toks = jnp.concatenate([toks, pad], axis=1)
    iix_arr = jnp.full((LANES,), iix, jnp.int32)

    mesh = plsc.VectorSubcoreMesh(core_axis_name="c", subcore_axis_name="s")
    counts_rm = pl.kernel(
        _sc_body,
        out_type=jax.ShapeDtypeStruct((BATCH, VOCAB), jnp.float32),
        mesh=mesh,
        scratch_types=[
            pltpu.VMEM((ROWS_PER_W, SEQ_PAD), jnp.int32),
            pltpu.VMEM((LANES,), jnp.int32),
            pltpu.VMEM((VOCAB,), jnp.float32),
        ],
        compiler_params=pltpu.CompilerParams(
            needs_layout_passes=False, use_tc_tiling_on_sc=True
        ),
    )(toks, iix_arr)
    return _to_batch_minor(counts_rm)


# speedup vs baseline: 1.3785x; 1.0263x over previous
import functools

import jax
import jax.numpy as jnp
from jax import lax
from jax.experimental import pallas as pl
from jax.experimental.pallas import tpu as pltpu
from jax.experimental.pallas import tpu_sc as plsc

VOCAB = 100000
BATCH = 1024
SEQ = 200
LANES = 16
GROUPS = (SEQ + LANES - 1) // LANES
SEQ_PAD = GROUPS * LANES
NUM_WORKERS = 32
ROWS_PER_W = BATCH // NUM_WORKERS
ZERO_UNROLL = 10

BGROUPS = BATCH // 128
VB = 12544
GV = (VOCAB + VB - 1) // VB
V8 = VOCAB // 8


def _sc_body(tok_hbm, iix_hbm, out_hbm, toks_v, iix_v, hist_v):
    c = lax.axis_index("c")
    s = lax.axis_index("s")
    wid = s * 2 + c
    base = wid * ROWS_PER_W

    pltpu.sync_copy(tok_hbm.at[pl.ds(base, ROWS_PER_W)], toks_v)
    pltpu.sync_copy(iix_hbm, iix_v)

    zeros_f = jnp.zeros((LANES,), jnp.float32)
    ones_f = jnp.ones((LANES,), jnp.float32)

    def _zero(i, carry):
        for u in range(ZERO_UNROLL):
            hist_v[pl.ds((i * ZERO_UNROLL + u) * LANES, LANES)] = zeros_f
        return carry

    lax.fori_loop(0, VOCAB // (LANES * ZERO_UNROLL), _zero, 0)

    iix16 = iix_v[...]

    def _row(r, carry):
        for g in range(GROUPS):
            idx = toks_v[r, pl.ds(g * LANES, LANES)]
            plsc.addupdate_scatter(hist_v, [idx], ones_f)
        plsc.store_scatter(hist_v, [iix16], zeros_f)
        pltpu.sync_copy(hist_v, out_hbm.at[base + r])
        for g in range(GROUPS):
            idx = toks_v[r, pl.ds(g * LANES, LANES)]
            plsc.store_scatter(hist_v, [idx], zeros_f)
        return carry

    lax.fori_loop(0, ROWS_PER_W, _row, 0)


def _tr_body(x_ref, o_ref):
    x = x_ref[...]
    o_ref[...] = x.T.reshape(VB // 8, 1, 8, 128)


def _to_batch_minor(counts_rm):
    counts4 = pl.pallas_call(
        _tr_body,
        grid=(BGROUPS, GV),
        in_specs=[pl.BlockSpec((128, VB), lambda i, j: (i, j))],
        out_specs=pl.BlockSpec((VB // 8, 1, 8, 128), lambda i, j: (j, i, 0, 0)),
        out_shape=jax.ShapeDtypeStruct((V8, BGROUPS, 8, 128), jnp.float32),
        compiler_params=pltpu.CompilerParams(
            dimension_semantics=("parallel", "parallel")
        ),
    )(counts_rm)
    return counts4.transpose(1, 3, 0, 2).reshape(BATCH, VOCAB)


@jax.jit
def kernel(inputs, ignore_index):
    iix = jnp.asarray(ignore_index, jnp.int32)
    toks = inputs.astype(jnp.int32)
    pad = jnp.full((BATCH, SEQ_PAD - SEQ), iix, jnp.int32)
    toks = jnp.concatenate([toks, pad], axis=1)
    iix_arr = jnp.full((LANES,), iix, jnp.int32)

    mesh = plsc.VectorSubcoreMesh(core_axis_name="c", subcore_axis_name="s")
    counts_rm = pl.kernel(
        _sc_body,
        out_type=jax.ShapeDtypeStruct((BATCH, VOCAB), jnp.float32),
        mesh=mesh,
        scratch_types=[
            pltpu.VMEM((ROWS_PER_W, SEQ_PAD), jnp.int32),
            pltpu.VMEM((LANES,), jnp.int32),
            pltpu.VMEM((VOCAB,), jnp.float32),
        ],
        compiler_params=pltpu.CompilerParams(
            needs_layout_passes=False, use_tc_tiling_on_sc=True
        ),
    )(toks, iix_arr)
    return _to_batch_minor(counts_rm)


# speedup vs baseline: 1.3965x; 1.0131x over previous
import functools

import jax
import jax.numpy as jnp
from jax import lax
from jax.experimental import pallas as pl
from jax.experimental.pallas import tpu as pltpu
from jax.experimental.pallas import tpu_sc as plsc

VOCAB = 100000
BATCH = 1024
SEQ = 200
LANES = 16
GROUPS = (SEQ + LANES - 1) // LANES
SEQ_PAD = GROUPS * LANES
NUM_WORKERS = 32
ROWS_PER_W = BATCH // NUM_WORKERS
ZERO_UNROLL = 10

BGROUPS = BATCH // 128
VB = 25088
GV = (VOCAB + VB - 1) // VB
V8 = VOCAB // 8


def _sc_body(tok_hbm, iix_hbm, out_hbm, toks_v, iix_v, hist_v):
    c = lax.axis_index("c")
    s = lax.axis_index("s")
    wid = s * 2 + c
    base = wid * ROWS_PER_W

    pltpu.sync_copy(tok_hbm.at[pl.ds(base, ROWS_PER_W)], toks_v)
    pltpu.sync_copy(iix_hbm, iix_v)

    zeros_f = jnp.zeros((LANES,), jnp.float32)
    ones_f = jnp.ones((LANES,), jnp.float32)

    def _zero(i, carry):
        for u in range(ZERO_UNROLL):
            hist_v[pl.ds((i * ZERO_UNROLL + u) * LANES, LANES)] = zeros_f
        return carry

    lax.fori_loop(0, VOCAB // (LANES * ZERO_UNROLL), _zero, 0)

    iix16 = iix_v[...]

    def _row(r, carry):
        for g in range(GROUPS):
            idx = toks_v[r, pl.ds(g * LANES, LANES)]
            plsc.addupdate_scatter(hist_v, [idx], ones_f)
        plsc.store_scatter(hist_v, [iix16], zeros_f)
        pltpu.sync_copy(hist_v, out_hbm.at[base + r])
        for g in range(GROUPS):
            idx = toks_v[r, pl.ds(g * LANES, LANES)]
            plsc.store_scatter(hist_v, [idx], zeros_f)
        return carry

    lax.fori_loop(0, ROWS_PER_W, _row, 0)


def _tr_body(x_ref, o_ref):
    x = x_ref[...]
    o_ref[...] = x.T.reshape(VB // 8, 1, 8, 128)


def _to_batch_minor(counts_rm):
    counts4 = pl.pallas_call(
        _tr_body,
        grid=(BGROUPS, GV),
        in_specs=[pl.BlockSpec((128, VB), lambda i, j: (i, j))],
        out_specs=pl.BlockSpec((VB // 8, 1, 8, 128), lambda i, j: (j, i, 0, 0)),
        out_shape=jax.ShapeDtypeStruct((V8, BGROUPS, 8, 128), jnp.float32),
        compiler_params=pltpu.CompilerParams(
            dimension_semantics=("parallel", "parallel")
        ),
    )(counts_rm)
    return counts4.transpose(1, 3, 0, 2).reshape(BATCH, VOCAB)


@jax.jit
def kernel(inputs, ignore_index):
    iix = jnp.asarray(ignore_index, jnp.int32)
    toks = inputs.astype(jnp.int32)
    pad = jnp.full((BATCH, SEQ_PAD - SEQ), iix, jnp.int32)
    toks = jnp.concatenate([toks, pad], axis=1)
    iix_arr = jnp.full((LANES,), iix, jnp.int32)

    mesh = plsc.VectorSubcoreMesh(core_axis_name="c", subcore_axis_name="s")
    counts_rm = pl.kernel(
        _sc_body,
        out_type=jax.ShapeDtypeStruct((BATCH, VOCAB), jnp.float32),
        mesh=mesh,
        scratch_types=[
            pltpu.VMEM((ROWS_PER_W, SEQ_PAD), jnp.int32),
            pltpu.VMEM((LANES,), jnp.int32),
            pltpu.VMEM((VOCAB,), jnp.float32),
        ],
        compiler_params=pltpu.CompilerParams(
            needs_layout_passes=False, use_tc_tiling_on_sc=True
        ),
    )(toks, iix_arr)
    return _to_batch_minor(counts_rm)
